# baseline (speedup 1.0000x reference)
import functools

import jax
import jax.numpy as jnp
from jax import lax
from jax.experimental import pallas as pl
from jax.experimental.pallas import tpu as pltpu
from jax.experimental.pallas import tpu_sc as plsc

_B = 16384
_F = 26
_D = 16
_NW = 32
_BPW = _B // _NW


def _fo_body(idx_hbm, w_hbm, out_hbm, idx_v, vals_v, acc_v, sem):
    wid = lax.axis_index("s") * 2 + lax.axis_index("c")
    pltpu.sync_copy(idx_hbm.at[wid], idx_v)
    pltpu.async_copy(w_hbm.at[idx_v], vals_v, sem).wait()
    for c in range(_BPW // 16):
        v = vals_v[pl.ds(c * 16, 16)]
        for f in range(1, _F):
            v = v + vals_v[pl.ds(f * _BPW + c * 16, 16)]
        acc_v[pl.ds(c * 16, 16)] = v
    pltpu.sync_copy(acc_v, out_hbm.at[pl.ds(wid * _BPW, _BPW)])


def _first_order(idx, w_flat):
    fo_kernel = functools.partial(
        pl.kernel,
        out_type=jax.ShapeDtypeStruct((_B,), jnp.float32),
        mesh=plsc.VectorSubcoreMesh(core_axis_name="c", subcore_axis_name="s"),
        scratch_types=[
            pltpu.VMEM((_F * _BPW,), jnp.int32),
            pltpu.VMEM((_F * _BPW,), jnp.float32),
            pltpu.VMEM((_BPW,), jnp.float32),
            pltpu.SemaphoreType.DMA,
        ],
    )(_fo_body)
    return fo_kernel(idx, w_flat)


def _so_body(x_ref, o_ref):
    x = x_ref[...]
    s = jnp.sum(x, axis=1)
    q = jnp.sum(jnp.sum(x * x, axis=1), axis=1)
    ssq = jnp.sum(s * s, axis=1)
    o_ref[...] = 0.5 * (ssq - q)


def _comb_body(a_ref, b_ref, o_ref):
    o_ref[...] = a_ref[...] + b_ref[...]


def kernel(sparse_inputs, embed_inputs, w):
    idx = sparse_inputs.T.reshape(_F, _NW, _BPW).transpose(1, 0, 2).reshape(_NW, _F * _BPW)
    fo = _first_order(idx, w.reshape(-1))

    blk = 256
    so = pl.pallas_call(
        _so_body,
        grid=(_B // blk,),
        in_specs=[pl.BlockSpec((blk, _F, _D), lambda i: (i, 0, 0))],
        out_specs=pl.BlockSpec((blk,), lambda i: (i,)),
        out_shape=jax.ShapeDtypeStruct((_B,), jnp.float32),
    )(embed_inputs)

    out = pl.pallas_call(
        _comb_body,
        in_specs=[
            pl.BlockSpec((_B,), lambda: (0,)),
            pl.BlockSpec((_B,), lambda: (0,)),
        ],
        out_specs=pl.BlockSpec((_B,), lambda: (0,)),
        out_shape=jax.ShapeDtypeStruct((_B,), jnp.float32),
    )(fo, so)
    return out.reshape(_B, 1)


# speedup vs baseline: 1.2624x; 1.2624x over previous
import functools

import jax
import jax.numpy as jnp
import numpy as np
from jax import lax
from jax.experimental import pallas as pl
from jax.experimental.pallas import tpu as pltpu
from jax.experimental.pallas import tpu_sc as plsc

_B = 16384
_F = 26
_D = 16
_NW = 32
_BPW = _B // _NW

_E = _F * _D
_GW = 6656
_GN = _B * _E // _GW
_KC = 512
_NK = _GW // _KC

_j = np.arange(_GW)
_k16 = _j // _E
_d16 = (_j % _E) % _D
_W1 = np.zeros((_GW, 16), np.float32)
_W1[_j, _k16] = 1.0
_W2 = np.zeros((_GW, 256), np.float32)
_W2[_j, _k16 * _D + _d16] = 1.0
_W3 = np.zeros((256, 16), np.float32)
_W3[np.arange(256), np.arange(256) // _D] = 1.0


def _fo_body(idx_hbm, w_hbm, out_hbm, idx_v, vals_v, acc_v, sem):
    wid = lax.axis_index("s") * 2 + lax.axis_index("c")
    pltpu.sync_copy(idx_hbm.at[wid], idx_v)
    pltpu.async_copy(w_hbm.at[idx_v], vals_v, sem).wait()
    for c in range(_BPW // 16):
        v = vals_v[pl.ds(c * 16, 16)]
        for f in range(1, _F):
            v = v + vals_v[pl.ds(f * _BPW + c * 16, 16)]
        acc_v[pl.ds(c * 16, 16)] = v
    pltpu.sync_copy(acc_v, out_hbm.at[pl.ds(wid * _BPW, _BPW)])


def _first_order(idx, w_flat):
    fo_kernel = functools.partial(
        pl.kernel,
        out_type=jax.ShapeDtypeStruct((_B,), jnp.float32),
        mesh=plsc.VectorSubcoreMesh(core_axis_name="c", subcore_axis_name="s"),
        scratch_types=[
            pltpu.VMEM((_F * _BPW,), jnp.int32),
            pltpu.VMEM((_F * _BPW,), jnp.float32),
            pltpu.VMEM((_BPW,), jnp.float32),
            pltpu.SemaphoreType.DMA,
        ],
    )(_fo_body)
    return fo_kernel(idx, w_flat)


def _so_body(x_ref, w1_ref, w2_ref, w3_ref, o_ref):
    blk = x_ref.shape[0]
    rc = 64
    for r in range(blk // rc):
        rs = pl.ds(r * rc, rc)
        s = jnp.zeros((rc, 256), jnp.float32)
        q = jnp.zeros((rc, 16), jnp.float32)
        for kk in range(_NK):
            ks = pl.ds(kk * _KC, _KC)
            xc = x_ref[rs, ks]
            s = s + jnp.dot(xc, w2_ref[ks, :], preferred_element_type=jnp.float32)
            q = q + jnp.dot(xc * xc, w1_ref[ks, :], preferred_element_type=jnp.float32)
        ssq = jnp.dot(s * s, w3_ref[...], preferred_element_type=jnp.float32)
        o_ref[rs, :] = 0.5 * (ssq - q)


def _comb_body(a_ref, b_ref, o_ref):
    o_ref[...] = a_ref[...] + b_ref[...]


def kernel(sparse_inputs, embed_inputs, w):
    idx = sparse_inputs.T.reshape(_F, _NW, _BPW).transpose(1, 0, 2).reshape(_NW, _F * _BPW)
    fo = _first_order(idx, w.reshape(-1))

    x = embed_inputs.reshape(_GN, _GW)
    blk = 128
    so = pl.pallas_call(
        _so_body,
        grid=(_GN // blk,),
        in_specs=[
            pl.BlockSpec((blk, _GW), lambda i: (i, 0)),
            pl.BlockSpec((_GW, 16), lambda i: (0, 0)),
            pl.BlockSpec((_GW, 256), lambda i: (0, 0)),
            pl.BlockSpec((256, 16), lambda i: (0, 0)),
        ],
        out_specs=pl.BlockSpec((blk, 16), lambda i: (i, 0)),
        out_shape=jax.ShapeDtypeStruct((_GN, 16), jnp.float32),
    )(x, jnp.asarray(_W1), jnp.asarray(_W2), jnp.asarray(_W3))

    out = pl.pallas_call(
        _comb_body,
        in_specs=[
            pl.BlockSpec((_B,), lambda: (0,)),
            pl.BlockSpec((_B,), lambda: (0,)),
        ],
        out_specs=pl.BlockSpec((_B,), lambda: (0,)),
        out_shape=jax.ShapeDtypeStruct((_B,), jnp.float32),
    )(fo, so.reshape(_B))
    return out.reshape(_B, 1)


# speedup vs baseline: 2.7885x; 2.2090x over previous
import functools

import jax
import jax.numpy as jnp
import numpy as np
from jax import lax
from jax.experimental import pallas as pl
from jax.experimental.pallas import tpu as pltpu
from jax.experimental.pallas import tpu_sc as plsc

_B = 16384
_F = 26
_D = 16
_NW = 32
_BPW = _B // _NW

_E = _F * _D


def _fo_body(idx_hbm, w_hbm, out_hbm, idx_v, vals_v, acc_v, sem):
    wid = lax.axis_index("s") * 2 + lax.axis_index("c")
    pltpu.sync_copy(idx_hbm.at[wid], idx_v)
    pltpu.async_copy(w_hbm.at[idx_v], vals_v, sem).wait()
    for c in range(_BPW // 16):
        v = vals_v[pl.ds(c * 16, 16)]
        for f in range(1, _F):
            v = v + vals_v[pl.ds(f * _BPW + c * 16, 16)]
        acc_v[pl.ds(c * 16, 16)] = v
    pltpu.sync_copy(acc_v, out_hbm.at[pl.ds(wid * _BPW, _BPW)])


def _first_order(idx, w_flat):
    fo_kernel = functools.partial(
        pl.kernel,
        out_type=jax.ShapeDtypeStruct((_B,), jnp.float32),
        mesh=plsc.VectorSubcoreMesh(core_axis_name="c", subcore_axis_name="s"),
        scratch_types=[
            pltpu.VMEM((_F * _BPW,), jnp.int32),
            pltpu.VMEM((_F * _BPW,), jnp.float32),
            pltpu.VMEM((_BPW,), jnp.float32),
            pltpu.SemaphoreType.DMA,
        ],
    )(_fo_body)
    return fo_kernel(idx, w_flat)


def _so_body(x_ref, m_ref, o_ref):
    blk = x_ref.shape[0]
    rc = 512
    for c in range(blk // rc):
        x = x_ref[pl.ds(c * rc, rc), :]
        q = jnp.sum(x * x, axis=1)
        s = jnp.dot(x, m_ref[...], preferred_element_type=jnp.float32)
        ssq = jnp.sum(s * s, axis=1)
        o_ref[pl.ds(c * rc, rc)] = 0.5 * (ssq - q)


def _comb_body(a_ref, b_ref, o_ref):
    o_ref[...] = a_ref[...] + b_ref[...]


def kernel(sparse_inputs, embed_inputs, w):
    idx = sparse_inputs.T.reshape(_F, _NW, _BPW).transpose(1, 0, 2).reshape(_NW, _F * _BPW)
    fo = _first_order(idx, w.reshape(-1))

    x = embed_inputs.reshape(_B, _E)
    m = jnp.tile(jnp.eye(_D, dtype=jnp.float32), (_F, 1))
    blk = 2048
    so = pl.pallas_call(
        _so_body,
        grid=(_B // blk,),
        in_specs=[
            pl.BlockSpec((blk, _E), lambda i: (i, 0)),
            pl.BlockSpec((_E, _D), lambda i: (0, 0)),
        ],
        out_specs=pl.BlockSpec((blk,), lambda i: (i,)),
        out_shape=jax.ShapeDtypeStruct((_B,), jnp.float32),
    )(x, m)

    out = pl.pallas_call(
        _comb_body,
        in_specs=[
            pl.BlockSpec((_B,), lambda: (0,)),
            pl.BlockSpec((_B,), lambda: (0,)),
        ],
        out_specs=pl.BlockSpec((_B,), lambda: (0,)),
        out_shape=jax.ShapeDtypeStruct((_B,), jnp.float32),
    )(fo, so)
    return out.reshape(_B, 1)


# speedup vs baseline: 3.5903x; 1.2875x over previous
import functools

import jax
import jax.numpy as jnp
import numpy as np
from jax import lax
from jax.experimental import pallas as pl
from jax.experimental.pallas import tpu as pltpu
from jax.experimental.pallas import tpu_sc as plsc

_B = 16384
_F = 26
_D = 16
_E = _F * _D
_NW = 32
_BPW = _B // _NW

_MS = (np.arange(_E)[None, :] % _D == np.arange(_D)[:, None]).astype(np.float32)


def _fo_body(idx_hbm, w_hbm, out_hbm, idx_v, vals_v, acc_v, sem):
    wid = lax.axis_index("s") * 2 + lax.axis_index("c")
    pltpu.sync_copy(idx_hbm.at[wid], idx_v)
    pltpu.async_copy(w_hbm.at[idx_v], vals_v, sem).wait()
    for c in range(_BPW // 16):
        v = vals_v[pl.ds(c * 16, 16)]
        for f in range(1, _F):
            v = v + vals_v[pl.ds(f * _BPW + c * 16, 16)]
        acc_v[pl.ds(c * 16, 16)] = v
    pltpu.sync_copy(acc_v, out_hbm.at[pl.ds(wid * _BPW, _BPW)])


def _first_order(idx, w):
    fo_kernel = functools.partial(
        pl.kernel,
        out_type=jax.ShapeDtypeStruct((_B,), jnp.float32),
        mesh=plsc.VectorSubcoreMesh(core_axis_name="c", subcore_axis_name="s"),
        scratch_types=[
            pltpu.VMEM((_F * _BPW,), jnp.int32),
            pltpu.VMEM((_F * _BPW,), jnp.float32),
            pltpu.VMEM((_BPW,), jnp.float32),
            pltpu.SemaphoreType.DMA,
        ],
    )(_fo_body)
    return fo_kernel(idx, w)


def _so_body(xt_ref, ms_ref, o_ref):
    bc = o_ref.shape[0]
    cw = 512
    gc = 104
    for c in range(bc // cw):
        cs = pl.ds(c * cw, cw)
        q = jnp.zeros((cw,), jnp.float32)
        s = jnp.zeros((_D, cw), jnp.float32)
        for g in range(_E // gc):
            xg = xt_ref[pl.ds(g * gc, gc), cs]
            q = q + jnp.sum(xg * xg, axis=0)
            s = s + jnp.dot(ms_ref[:, pl.ds(g * gc, gc)], xg,
                            preferred_element_type=jnp.float32)
        ssq = jnp.sum(s * s, axis=0)
        o_ref[cs] = 0.5 * (ssq - q)


def _comb_body(a_ref, b_ref, o_ref):
    o_ref[...] = a_ref[...] + b_ref[...]


def kernel(sparse_inputs, embed_inputs, w):
    idx = sparse_inputs.T.reshape(_F, _NW, _BPW).transpose(1, 0, 2).reshape(_NW, _F * _BPW)
    fo = _first_order(idx, w.T.reshape(-1))

    xt = embed_inputs.reshape(_B, _E).T
    blk = 2048
    so = pl.pallas_call(
        _so_body,
        grid=(_B // blk,),
        in_specs=[
            pl.BlockSpec((_E, blk), lambda i: (0, i)),
            pl.BlockSpec((_D, _E), lambda i: (0, 0)),
        ],
        out_specs=pl.BlockSpec((blk,), lambda i: (i,)),
        out_shape=jax.ShapeDtypeStruct((_B,), jnp.float32),
    )(xt, jnp.asarray(_MS))

    out = pl.pallas_call(
        _comb_body,
        in_specs=[
            pl.BlockSpec((_B,), lambda: (0,)),
            pl.BlockSpec((_B,), lambda: (0,)),
        ],
        out_specs=pl.BlockSpec((_B,), lambda: (0,)),
        out_shape=jax.ShapeDtypeStruct((_B,), jnp.float32),
    )(fo, so)
    return out.reshape(_B, 1)
